# baseline (speedup 1.0000x reference)
import functools

import jax
import jax.numpy as jnp
from jax import lax
from jax.experimental import pallas as pl
from jax.experimental.pallas import tpu as pltpu
from jax.experimental.pallas import tpu_sc as plsc

NPOINT = 2048
NSAMPLE = 32
RADIUS = 0.2
BN_EPS = 1e-5

_NC = 2
_NS = 16
_L = 16
_KBUF = 4


def _fps_body(xyzt_ref, xyzn_ref, out_ref):
    X0, Y0, Z0 = xyzt_ref[0, 0], xyzt_ref[0, 1], xyzt_ref[0, 2]
    X1, Y1, Z1 = xyzt_ref[1, 0], xyzt_ref[1, 1], xyzt_ref[1, 2]
    iota2df = (lax.broadcasted_iota(jnp.int32, (64, 128), 0) * 128
               + lax.broadcasted_iota(jnp.int32, (64, 128), 1)
               ).astype(jnp.float32)
    out_ref[0, 0:1, :] = xyzn_ref[0, 0:1, :]
    out_ref[1, 0:1, :] = xyzn_ref[1, 0:1, :]
    bigf = jnp.float32(1e9)

    def body(i, carry):
        d0, d1, l0, l1 = carry
        c0 = xyzn_ref[0, pl.ds(l0, 1), :]
        c1 = xyzn_ref[1, pl.ds(l1, 1), :]
        dx0 = X0 - c0[0, 0]
        dy0 = Y0 - c0[0, 1]
        dz0 = Z0 - c0[0, 2]
        dx1 = X1 - c1[0, 0]
        dy1 = Y1 - c1[0, 1]
        dz1 = Z1 - c1[0, 2]
        d0 = jnp.minimum(d0, (dx0 * dx0 + dy0 * dy0) + dz0 * dz0)
        d1 = jnp.minimum(d1, (dx1 * dx1 + dy1 * dy1) + dz1 * dz1)
        n0 = jnp.argmax(d0.reshape(-1)).astype(jnp.int32)
        n1 = jnp.argmax(d1.reshape(-1)).astype(jnp.int32)
        out_ref[0, pl.ds(i, 1), :] = xyzn_ref[0, pl.ds(n0, 1), :]
        out_ref[1, pl.ds(i, 1), :] = xyzn_ref[1, pl.ds(n1, 1), :]
        return d0, d1, n0, n1

    dists0 = jnp.full((64, 128), 1e10, dtype=jnp.float32)
    lax.fori_loop(1, NPOINT, body,
                  (dists0, dists0, jnp.int32(0), jnp.int32(0)))


def _fps(xyz_t3, xyz):
    B = xyz.shape[0]
    return pl.pallas_call(
        _fps_body,
        out_shape=jax.ShapeDtypeStruct((B, NPOINT, 3), jnp.float32),
    )(xyz_t3, xyz)


def _gtable_body(xyz_ref, f_ref, w1_ref, b1_ref, out_ref):
    w1 = w1_ref[...]
    gx = jnp.dot(xyz_ref[0], w1[:, 0:3].T, preferred_element_type=jnp.float32)
    gf = lax.dot_general(f_ref[0], w1[:, 3:35],
                         (((0,), (1,)), ((), ())),
                         preferred_element_type=jnp.float32)
    out_ref[0] = (gx + gf) + b1_ref[...]


def _gtable(xyz, features, W1, b1):
    B, N, _ = xyz.shape
    return pl.pallas_call(
        _gtable_body,
        grid=(B, 8),
        in_specs=[
            pl.BlockSpec((1, 1024, 3), lambda b, j: (b, j, 0)),
            pl.BlockSpec((1, 32, 1024), lambda b, j: (b, 0, j)),
            pl.BlockSpec((64, 35), lambda b, j: (0, 0)),
            pl.BlockSpec((1, 64), lambda b, j: (0, 0)),
        ],
        out_specs=pl.BlockSpec((1, 1024, 64), lambda b, j: (b, j, 0)),
        out_shape=jax.ShapeDtypeStruct((B, N, 64), jnp.float32),
    )(xyz, features, W1, b1.reshape(1, 64))


def _ballgather(xyz_flat, new_xyz_flat, g_flat):
    B = xyz_flat.shape[0] // (3 * 8192)
    n_rows = B * NPOINT
    rows_per_w = n_rows // (_NC * _NS)
    w_per_b = _NS * _NC // B
    r2 = jnp.float32(RADIUS * RADIUS)
    n_steps = 8192 // _L
    mesh = plsc.VectorSubcoreMesh(core_axis_name="c", subcore_axis_name="s")

    @functools.partial(
        pl.kernel,
        out_type=jax.ShapeDtypeStruct((n_rows, NSAMPLE, 64), jnp.float32),
        mesh=mesh,
        compiler_params=pltpu.CompilerParams(needs_layout_passes=False,
                                             use_tc_tiling_on_sc=False),
        scratch_types=[
            pltpu.VMEM((8192,), jnp.float32),
            pltpu.VMEM((8192,), jnp.float32),
            pltpu.VMEM((8192,), jnp.float32),
            pltpu.VMEM((rows_per_w + _L,), jnp.float32),
            pltpu.VMEM((rows_per_w + _L,), jnp.float32),
            pltpu.VMEM((rows_per_w + _L,), jnp.float32),
            pltpu.VMEM((NSAMPLE + _L,), jnp.int32),
            [pltpu.VMEM((NSAMPLE,), jnp.int32) for _ in range(_KBUF)],
            [pltpu.VMEM((NSAMPLE, 64), jnp.float32) for _ in range(_KBUF)],
            [pltpu.SemaphoreType.DMA for _ in range(_KBUF)],
            [pltpu.SemaphoreType.DMA for _ in range(_KBUF)],
        ],
    )
    def k(xyz_hbm, q_hbm, g_hbm, out_hbm, xx_v, xy_v, xz_v, qx_v, qy_v, qz_v,
          buf_v, idx_vs, rows_vs, gsems, wsems):
        wid = lax.axis_index("s") * _NC + lax.axis_index("c")
        b = wid // w_per_b
        s0 = (wid % w_per_b) * rows_per_w
        pltpu.sync_copy(xyz_hbm.at[pl.ds((b * 3 + 0) * 8192, 8192)], xx_v)
        pltpu.sync_copy(xyz_hbm.at[pl.ds((b * 3 + 1) * 8192, 8192)], xy_v)
        pltpu.sync_copy(xyz_hbm.at[pl.ds((b * 3 + 2) * 8192, 8192)], xz_v)
        pltpu.sync_copy(
            q_hbm.at[pl.ds((b * 3 + 0) * NPOINT + s0, rows_per_w)],
            qx_v.at[pl.ds(0, rows_per_w)])
        pltpu.sync_copy(
            q_hbm.at[pl.ds((b * 3 + 1) * NPOINT + s0, rows_per_w)],
            qy_v.at[pl.ds(0, rows_per_w)])
        pltpu.sync_copy(
            q_hbm.at[pl.ds((b * 3 + 2) * NPOINT + s0, rows_per_w)],
            qz_v.at[pl.ds(0, rows_per_w)])
        base = b * 8192
        wbase = wid * rows_per_w
        iota = lax.iota(jnp.int32, _L)
        big = jnp.int32(1 << 30)

        def scan_row(r, idx_v):
            qx = jnp.full((_L,), qx_v[pl.ds(r, _L)][0], jnp.float32)
            qy = jnp.full((_L,), qy_v[pl.ds(r, _L)][0], jnp.float32)
            qz = jnp.full((_L,), qz_v[pl.ds(r, _L)][0], jnp.float32)

            def cond(st):
                return jnp.logical_and(st[1] < NSAMPLE, st[0] < n_steps)

            def wbody(st):
                step, cnt = st
                off = step * _L
                px = xx_v[pl.ds(off, _L)]
                py = xy_v[pl.ds(off, _L)]
                pz = xz_v[pl.ds(off, _L)]
                dx = px - qx
                dy = py - qy
                dz = pz - qz
                d2 = (dx * dx + dy * dy) + dz * dz
                msk = d2 < r2
                cols = iota + (off + base)
                plsc.store_compressed(buf_v.at[pl.ds(cnt, _L)], cols, mask=msk)
                nhit = plsc.all_reduce_population_count(msk)[0]
                return (step + 1, cnt + nhit)

            st = lax.while_loop(cond, wbody, (jnp.int32(0), jnp.int32(0)))
            cnt = st[1]
            first = buf_v[pl.ds(0, _L)][0]
            for j in range(NSAMPLE // _L):
                v = buf_v[pl.ds(_L * j, _L)]
                lane = iota + _L * j
                idx_v[pl.ds(_L * j, _L)] = jnp.where(lane < cnt, v, first)

        for kk in range(_KBUF):
            scan_row(jnp.int32(kk), idx_vs[kk])
            pltpu.async_copy(g_hbm.at[idx_vs[kk]], rows_vs[kk], gsems[kk])

        def outer_body(o, _):
            for kk in range(_KBUF):
                row = o * _KBUF + kk
                pltpu.make_async_copy(
                    g_hbm.at[idx_vs[kk]], rows_vs[kk], gsems[kk]).wait()
                wr = pltpu.async_copy(
                    rows_vs[kk], out_hbm.at[wbase + row - _KBUF], wsems[kk])
                scan_row(row, idx_vs[kk])
                wr.wait()
                pltpu.async_copy(g_hbm.at[idx_vs[kk]], rows_vs[kk], gsems[kk])
            return 0

        lax.fori_loop(1, rows_per_w // _KBUF, outer_body, 0)
        for kk in range(_KBUF):
            row = rows_per_w - _KBUF + kk
            pltpu.make_async_copy(
                g_hbm.at[idx_vs[kk]], rows_vs[kk], gsems[kk]).wait()
            pltpu.sync_copy(rows_vs[kk], out_hbm.at[wbase + row])

    return k(xyz_flat, new_xyz_flat, g_flat)


def _mlp_body(g_ref, q_ref, w1_ref, w2_ref, b2_ref, w3_ref, b3_ref, out_ref):
    s = jnp.sqrt(jnp.float32(1.0 + BN_EPS))
    g = g_ref[0]
    q = q_ref[0]
    w1x = w1_ref[...][:, 0:3]
    t = jnp.dot(q, w1x.T, preferred_element_type=jnp.float32)
    h1 = jax.nn.relu((g - t[:, None, :]) / s)
    h1 = h1.reshape(256 * NSAMPLE, 64)
    y2 = jnp.dot(h1, w2_ref[...].T, preferred_element_type=jnp.float32)
    h2 = jax.nn.relu((y2 + b2_ref[...]) / s)
    y3 = jnp.dot(h2, w3_ref[...].T, preferred_element_type=jnp.float32)
    h3 = jax.nn.relu((y3 + b3_ref[...]) / s)
    m = jnp.max(h3.reshape(256, NSAMPLE, 128), axis=1)
    out_ref[0] = m.T


def _mlp(gathered, new_xyz, W1, W2, b2, W3, b3):
    B = gathered.shape[0]
    return pl.pallas_call(
        _mlp_body,
        grid=(B, NPOINT // 256),
        in_specs=[
            pl.BlockSpec((1, 256, NSAMPLE, 64), lambda b, j: (b, j, 0, 0)),
            pl.BlockSpec((1, 256, 3), lambda b, j: (b, j, 0)),
            pl.BlockSpec((64, 35), lambda b, j: (0, 0)),
            pl.BlockSpec((64, 64), lambda b, j: (0, 0)),
            pl.BlockSpec((1, 64), lambda b, j: (0, 0)),
            pl.BlockSpec((128, 64), lambda b, j: (0, 0)),
            pl.BlockSpec((1, 128), lambda b, j: (0, 0)),
        ],
        out_specs=pl.BlockSpec((1, 128, 256), lambda b, j: (b, 0, j)),
        out_shape=jax.ShapeDtypeStruct((B, 128, NPOINT), jnp.float32),
    )(gathered, new_xyz, W1, W2, b2.reshape(1, 64), W3, b3.reshape(1, 128))


def kernel(xyz, features, W1, b1, W2, b2, W3, b3):
    B, N, _ = xyz.shape
    xyz_soa = jnp.transpose(xyz, (0, 2, 1))
    xyz_t3 = xyz_soa.reshape(B, 3, N // 128, 128)

    new_xyz = _fps(xyz_t3, xyz)
    G = _gtable(xyz, features, W1, b1)
    new_xyz_soa = jnp.transpose(new_xyz, (0, 2, 1))
    gathered = _ballgather(xyz_soa.reshape(B * 3 * N),
                           new_xyz_soa.reshape(B * 3 * NPOINT),
                           G.reshape(B * N, 64))
    gathered = gathered.reshape(B, NPOINT, NSAMPLE, 64)
    new_features = _mlp(gathered, new_xyz, W1, W2, b2, W3, b3)
    return new_xyz, new_features


# speedup vs baseline: 1.0607x; 1.0607x over previous
import functools

import jax
import jax.numpy as jnp
from jax import lax
from jax.experimental import pallas as pl
from jax.experimental.pallas import tpu as pltpu
from jax.experimental.pallas import tpu_sc as plsc

NPOINT = 2048
NSAMPLE = 32
RADIUS = 0.2
BN_EPS = 1e-5

_NC = 2
_NS = 16
_L = 16
_KBUF = 4


def _fps_body(xyzt_ref, xyzn_ref, out_ref):
    X0, Y0, Z0 = xyzt_ref[0, 0], xyzt_ref[0, 1], xyzt_ref[0, 2]
    X1, Y1, Z1 = xyzt_ref[1, 0], xyzt_ref[1, 1], xyzt_ref[1, 2]
    iota2df = (lax.broadcasted_iota(jnp.int32, (64, 128), 0) * 128
               + lax.broadcasted_iota(jnp.int32, (64, 128), 1)
               ).astype(jnp.float32)
    out_ref[0, 0:1, :] = xyzn_ref[0, 0:1, :]
    out_ref[1, 0:1, :] = xyzn_ref[1, 0:1, :]
    bigf = jnp.float32(1e9)

    def body(i, carry):
        d0, d1, l0, l1 = carry
        c0 = xyzn_ref[0, pl.ds(l0, 1), :]
        c1 = xyzn_ref[1, pl.ds(l1, 1), :]
        dx0 = X0 - c0[0, 0]
        dy0 = Y0 - c0[0, 1]
        dz0 = Z0 - c0[0, 2]
        dx1 = X1 - c1[0, 0]
        dy1 = Y1 - c1[0, 1]
        dz1 = Z1 - c1[0, 2]
        d0 = jnp.minimum(d0, (dx0 * dx0 + dy0 * dy0) + dz0 * dz0)
        d1 = jnp.minimum(d1, (dx1 * dx1 + dy1 * dy1) + dz1 * dz1)
        m0 = jnp.max(d0, axis=(0, 1), keepdims=True)
        m1 = jnp.max(d1, axis=(0, 1), keepdims=True)
        n0 = jnp.min(jnp.where(d0 == m0, iota2df, bigf)).astype(jnp.int32)
        n1 = jnp.min(jnp.where(d1 == m1, iota2df, bigf)).astype(jnp.int32)
        out_ref[0, pl.ds(i, 1), :] = xyzn_ref[0, pl.ds(n0, 1), :]
        out_ref[1, pl.ds(i, 1), :] = xyzn_ref[1, pl.ds(n1, 1), :]
        return d0, d1, n0, n1

    dists0 = jnp.full((64, 128), 1e10, dtype=jnp.float32)
    lax.fori_loop(1, NPOINT, body,
                  (dists0, dists0, jnp.int32(0), jnp.int32(0)))


def _fps(xyz_t3, xyz):
    B = xyz.shape[0]
    return pl.pallas_call(
        _fps_body,
        out_shape=jax.ShapeDtypeStruct((B, NPOINT, 3), jnp.float32),
    )(xyz_t3, xyz)


def _gtable_body(xyz_ref, f_ref, w1_ref, b1_ref, out_ref):
    w1 = w1_ref[...]
    gx = jnp.dot(xyz_ref[0], w1[:, 0:3].T, preferred_element_type=jnp.float32)
    gf = lax.dot_general(f_ref[0], w1[:, 3:35],
                         (((0,), (1,)), ((), ())),
                         preferred_element_type=jnp.float32)
    out_ref[0] = (gx + gf) + b1_ref[...]


def _gtable(xyz, features, W1, b1):
    B, N, _ = xyz.shape
    return pl.pallas_call(
        _gtable_body,
        grid=(B, 8),
        in_specs=[
            pl.BlockSpec((1, 1024, 3), lambda b, j: (b, j, 0)),
            pl.BlockSpec((1, 32, 1024), lambda b, j: (b, 0, j)),
            pl.BlockSpec((64, 35), lambda b, j: (0, 0)),
            pl.BlockSpec((1, 64), lambda b, j: (0, 0)),
        ],
        out_specs=pl.BlockSpec((1, 1024, 64), lambda b, j: (b, j, 0)),
        out_shape=jax.ShapeDtypeStruct((B, N, 64), jnp.float32),
    )(xyz, features, W1, b1.reshape(1, 64))


def _ballgather(xyz_flat, new_xyz_flat, g_flat):
    B = xyz_flat.shape[0] // (3 * 8192)
    n_rows = B * NPOINT
    rows_per_w = n_rows // (_NC * _NS)
    w_per_b = _NS * _NC // B
    r2 = jnp.float32(RADIUS * RADIUS)
    n_steps = 8192 // _L
    mesh = plsc.VectorSubcoreMesh(core_axis_name="c", subcore_axis_name="s")

    @functools.partial(
        pl.kernel,
        out_type=jax.ShapeDtypeStruct((n_rows, NSAMPLE, 64), jnp.float32),
        mesh=mesh,
        compiler_params=pltpu.CompilerParams(needs_layout_passes=False,
                                             use_tc_tiling_on_sc=False),
        scratch_types=[
            pltpu.VMEM((8192,), jnp.float32),
            pltpu.VMEM((8192,), jnp.float32),
            pltpu.VMEM((8192,), jnp.float32),
            pltpu.VMEM((rows_per_w + _L,), jnp.float32),
            pltpu.VMEM((rows_per_w + _L,), jnp.float32),
            pltpu.VMEM((rows_per_w + _L,), jnp.float32),
            pltpu.VMEM((NSAMPLE + _L,), jnp.int32),
            [pltpu.VMEM((NSAMPLE,), jnp.int32) for _ in range(_KBUF)],
            [pltpu.VMEM((NSAMPLE, 64), jnp.float32) for _ in range(_KBUF)],
            [pltpu.SemaphoreType.DMA for _ in range(_KBUF)],
            [pltpu.SemaphoreType.DMA for _ in range(_KBUF)],
        ],
    )
    def k(xyz_hbm, q_hbm, g_hbm, out_hbm, xx_v, xy_v, xz_v, qx_v, qy_v, qz_v,
          buf_v, idx_vs, rows_vs, gsems, wsems):
        wid = lax.axis_index("s") * _NC + lax.axis_index("c")
        b = wid // w_per_b
        s0 = (wid % w_per_b) * rows_per_w
        pltpu.sync_copy(xyz_hbm.at[pl.ds((b * 3 + 0) * 8192, 8192)], xx_v)
        pltpu.sync_copy(xyz_hbm.at[pl.ds((b * 3 + 1) * 8192, 8192)], xy_v)
        pltpu.sync_copy(xyz_hbm.at[pl.ds((b * 3 + 2) * 8192, 8192)], xz_v)
        pltpu.sync_copy(
            q_hbm.at[pl.ds((b * 3 + 0) * NPOINT + s0, rows_per_w)],
            qx_v.at[pl.ds(0, rows_per_w)])
        pltpu.sync_copy(
            q_hbm.at[pl.ds((b * 3 + 1) * NPOINT + s0, rows_per_w)],
            qy_v.at[pl.ds(0, rows_per_w)])
        pltpu.sync_copy(
            q_hbm.at[pl.ds((b * 3 + 2) * NPOINT + s0, rows_per_w)],
            qz_v.at[pl.ds(0, rows_per_w)])
        base = b * 8192
        wbase = wid * rows_per_w
        iota = lax.iota(jnp.int32, _L)
        big = jnp.int32(1 << 30)

        def scan_row(r, idx_v):
            qx = jnp.full((_L,), qx_v[pl.ds(r, _L)][0], jnp.float32)
            qy = jnp.full((_L,), qy_v[pl.ds(r, _L)][0], jnp.float32)
            qz = jnp.full((_L,), qz_v[pl.ds(r, _L)][0], jnp.float32)

            def cond(st):
                return jnp.logical_and(st[1] < NSAMPLE, st[0] < n_steps)

            def wbody(st):
                step, cnt = st
                off = step * _L
                px = xx_v[pl.ds(off, _L)]
                py = xy_v[pl.ds(off, _L)]
                pz = xz_v[pl.ds(off, _L)]
                dx = px - qx
                dy = py - qy
                dz = pz - qz
                d2 = (dx * dx + dy * dy) + dz * dz
                msk = d2 < r2
                cols = iota + (off + base)
                plsc.store_compressed(buf_v.at[pl.ds(cnt, _L)], cols, mask=msk)
                nhit = plsc.all_reduce_population_count(msk)[0]
                return (step + 1, cnt + nhit)

            st = lax.while_loop(cond, wbody, (jnp.int32(0), jnp.int32(0)))
            cnt = st[1]
            first = buf_v[pl.ds(0, _L)][0]
            for j in range(NSAMPLE // _L):
                v = buf_v[pl.ds(_L * j, _L)]
                lane = iota + _L * j
                idx_v[pl.ds(_L * j, _L)] = jnp.where(lane < cnt, v, first)

        for kk in range(_KBUF):
            scan_row(jnp.int32(kk), idx_vs[kk])
            pltpu.async_copy(g_hbm.at[idx_vs[kk]], rows_vs[kk], gsems[kk])

        def outer_body(o, _):
            for kk in range(_KBUF):
                row = o * _KBUF + kk
                pltpu.make_async_copy(
                    g_hbm.at[idx_vs[kk]], rows_vs[kk], gsems[kk]).wait()
                wr = pltpu.async_copy(
                    rows_vs[kk], out_hbm.at[wbase + row - _KBUF], wsems[kk])
                scan_row(row, idx_vs[kk])
                wr.wait()
                pltpu.async_copy(g_hbm.at[idx_vs[kk]], rows_vs[kk], gsems[kk])
            return 0

        lax.fori_loop(1, rows_per_w // _KBUF, outer_body, 0)
        for kk in range(_KBUF):
            row = rows_per_w - _KBUF + kk
            pltpu.make_async_copy(
                g_hbm.at[idx_vs[kk]], rows_vs[kk], gsems[kk]).wait()
            pltpu.sync_copy(rows_vs[kk], out_hbm.at[wbase + row])

    return k(xyz_flat, new_xyz_flat, g_flat)


def _mlp_body(g_ref, q_ref, w1_ref, w2_ref, b2_ref, w3_ref, b3_ref, out_ref):
    s = jnp.sqrt(jnp.float32(1.0 + BN_EPS))
    g = g_ref[...]
    q = q_ref[...]
    w1x = w1_ref[...][:, 0:3]
    t = jnp.dot(q, w1x.T, preferred_element_type=jnp.float32)
    h1 = jax.nn.relu((g - t[:, None, :]) / s)
    h1 = h1.reshape(256 * NSAMPLE, 64)
    y2 = jnp.dot(h1, w2_ref[...].T, preferred_element_type=jnp.float32)
    h2 = jax.nn.relu((y2 + b2_ref[...]) / s)
    y3 = jnp.dot(h2, w3_ref[...].T, preferred_element_type=jnp.float32)
    h3 = jax.nn.relu((y3 + b3_ref[...]) / s)
    m = jnp.max(h3.reshape(256, NSAMPLE, 128), axis=1)
    out_ref[0] = m.T


def _mlp(gathered, new_xyz_flat, W1, W2, b2, W3, b3, B):
    return pl.pallas_call(
        _mlp_body,
        grid=(B, NPOINT // 256),
        in_specs=[
            pl.BlockSpec((256, NSAMPLE, 64),
                         lambda b, j: (b * (NPOINT // 256) + j, 0, 0)),
            pl.BlockSpec((256, 3), lambda b, j: (b * (NPOINT // 256) + j, 0)),
            pl.BlockSpec((64, 35), lambda b, j: (0, 0)),
            pl.BlockSpec((64, 64), lambda b, j: (0, 0)),
            pl.BlockSpec((1, 64), lambda b, j: (0, 0)),
            pl.BlockSpec((128, 64), lambda b, j: (0, 0)),
            pl.BlockSpec((1, 128), lambda b, j: (0, 0)),
        ],
        out_specs=pl.BlockSpec((1, 128, 256), lambda b, j: (b, 0, j)),
        out_shape=jax.ShapeDtypeStruct((B, 128, NPOINT), jnp.float32),
    )(gathered, new_xyz_flat, W1, W2, b2.reshape(1, 64), W3,
      b3.reshape(1, 128))


def kernel(xyz, features, W1, b1, W2, b2, W3, b3):
    B, N, _ = xyz.shape
    xyz_soa = jnp.transpose(xyz, (0, 2, 1))
    xyz_t3 = xyz_soa.reshape(B, 3, N // 128, 128)

    new_xyz = _fps(xyz_t3, xyz)
    G = _gtable(xyz, features, W1, b1)
    new_xyz_soa = jnp.transpose(new_xyz, (0, 2, 1))
    gathered = _ballgather(xyz_soa.reshape(B * 3 * N),
                           new_xyz_soa.reshape(B * 3 * NPOINT),
                           G.reshape(B * N, 64))
    new_features = _mlp(gathered, new_xyz.reshape(B * NPOINT, 3),
                        W1, W2, b2, W3, b3, B)
    return new_xyz, new_features


# speedup vs baseline: 1.1437x; 1.0783x over previous
import functools

import jax
import jax.numpy as jnp
from jax import lax
from jax.experimental import pallas as pl
from jax.experimental.pallas import tpu as pltpu
from jax.experimental.pallas import tpu_sc as plsc

NPOINT = 2048
NSAMPLE = 32
RADIUS = 0.2
BN_EPS = 1e-5

_NC = 2
_NS = 16
_L = 16
_KBUF = 4


def _fps_body(xyzt_ref, xyzn_ref, out_ref):
    X0, Y0, Z0 = xyzt_ref[0, 0], xyzt_ref[0, 1], xyzt_ref[0, 2]
    X1, Y1, Z1 = xyzt_ref[1, 0], xyzt_ref[1, 1], xyzt_ref[1, 2]
    iota2df = (lax.broadcasted_iota(jnp.int32, (64, 128), 0) * 128
               + lax.broadcasted_iota(jnp.int32, (64, 128), 1)
               ).astype(jnp.float32)
    out_ref[0, 0:1, :] = xyzn_ref[0, 0:1, :]
    out_ref[1, 0:1, :] = xyzn_ref[1, 0:1, :]
    bigf = jnp.float32(1e9)

    def body(i, carry):
        d0, d1, l0, l1 = carry
        c0 = xyzn_ref[0, pl.ds(l0, 1), :]
        c1 = xyzn_ref[1, pl.ds(l1, 1), :]
        dx0 = X0 - c0[0, 0]
        dy0 = Y0 - c0[0, 1]
        dz0 = Z0 - c0[0, 2]
        dx1 = X1 - c1[0, 0]
        dy1 = Y1 - c1[0, 1]
        dz1 = Z1 - c1[0, 2]
        d0 = jnp.minimum(d0, (dx0 * dx0 + dy0 * dy0) + dz0 * dz0)
        d1 = jnp.minimum(d1, (dx1 * dx1 + dy1 * dy1) + dz1 * dz1)
        m0 = jnp.max(d0, axis=(0, 1), keepdims=True)
        m1 = jnp.max(d1, axis=(0, 1), keepdims=True)
        n0 = jnp.min(jnp.where(d0 == m0, iota2df, bigf)).astype(jnp.int32)
        n1 = jnp.min(jnp.where(d1 == m1, iota2df, bigf)).astype(jnp.int32)
        out_ref[0, pl.ds(i, 1), :] = xyzn_ref[0, pl.ds(n0, 1), :]
        out_ref[1, pl.ds(i, 1), :] = xyzn_ref[1, pl.ds(n1, 1), :]
        return d0, d1, n0, n1

    dists0 = jnp.full((64, 128), 1e10, dtype=jnp.float32)
    lax.fori_loop(1, NPOINT, body,
                  (dists0, dists0, jnp.int32(0), jnp.int32(0)))


def _fps(xyz_t3, xyz):
    B = xyz.shape[0]
    return pl.pallas_call(
        _fps_body,
        out_shape=jax.ShapeDtypeStruct((B, NPOINT, 3), jnp.float32),
    )(xyz_t3, xyz)


def _gtable_body(xyz_ref, f_ref, w1_ref, b1_ref, out_ref):
    w1 = w1_ref[...]
    gx = jnp.dot(xyz_ref[0], w1[:, 0:3].T, preferred_element_type=jnp.float32)
    gf = lax.dot_general(f_ref[0], w1[:, 3:35],
                         (((0,), (1,)), ((), ())),
                         preferred_element_type=jnp.float32)
    out_ref[0] = (gx + gf) + b1_ref[...]


def _gtable(xyz, features, W1, b1):
    B, N, _ = xyz.shape
    return pl.pallas_call(
        _gtable_body,
        grid=(B, 8),
        in_specs=[
            pl.BlockSpec((1, 1024, 3), lambda b, j: (b, j, 0)),
            pl.BlockSpec((1, 32, 1024), lambda b, j: (b, 0, j)),
            pl.BlockSpec((64, 35), lambda b, j: (0, 0)),
            pl.BlockSpec((1, 64), lambda b, j: (0, 0)),
        ],
        out_specs=pl.BlockSpec((1, 1024, 64), lambda b, j: (b, j, 0)),
        out_shape=jax.ShapeDtypeStruct((B, N, 64), jnp.float32),
    )(xyz, features, W1, b1.reshape(1, 64))


def _ballgather(xyz_flat, new_xyz_flat, g_flat):
    B = xyz_flat.shape[0] // (3 * 8192)
    n_rows = B * NPOINT
    rows_per_w = n_rows // (_NC * _NS)
    w_per_b = _NS * _NC // B
    r2 = jnp.float32(RADIUS * RADIUS)
    n_steps = 8192 // _L
    mesh = plsc.VectorSubcoreMesh(core_axis_name="c", subcore_axis_name="s")

    @functools.partial(
        pl.kernel,
        out_type=jax.ShapeDtypeStruct((n_rows, NSAMPLE, 64), jnp.float32),
        mesh=mesh,
        compiler_params=pltpu.CompilerParams(needs_layout_passes=False,
                                             use_tc_tiling_on_sc=False),
        scratch_types=[
            pltpu.VMEM((8192,), jnp.float32),
            pltpu.VMEM((8192,), jnp.float32),
            pltpu.VMEM((8192,), jnp.float32),
            pltpu.VMEM((rows_per_w + _L,), jnp.float32),
            pltpu.VMEM((rows_per_w + _L,), jnp.float32),
            pltpu.VMEM((rows_per_w + _L,), jnp.float32),
            pltpu.VMEM((NSAMPLE + 2 * _L,), jnp.int32),
            [pltpu.VMEM((NSAMPLE,), jnp.int32) for _ in range(_KBUF)],
            [pltpu.VMEM((NSAMPLE, 64), jnp.float32) for _ in range(_KBUF)],
            [pltpu.SemaphoreType.DMA for _ in range(_KBUF)],
            [pltpu.SemaphoreType.DMA for _ in range(_KBUF)],
        ],
    )
    def k(xyz_hbm, q_hbm, g_hbm, out_hbm, xx_v, xy_v, xz_v, qx_v, qy_v, qz_v,
          buf_v, idx_vs, rows_vs, gsems, wsems):
        wid = lax.axis_index("s") * _NC + lax.axis_index("c")
        b = wid // w_per_b
        s0 = (wid % w_per_b) * rows_per_w
        pltpu.sync_copy(xyz_hbm.at[pl.ds((b * 3 + 0) * 8192, 8192)], xx_v)
        pltpu.sync_copy(xyz_hbm.at[pl.ds((b * 3 + 1) * 8192, 8192)], xy_v)
        pltpu.sync_copy(xyz_hbm.at[pl.ds((b * 3 + 2) * 8192, 8192)], xz_v)
        pltpu.sync_copy(
            q_hbm.at[pl.ds((b * 3 + 0) * NPOINT + s0, rows_per_w)],
            qx_v.at[pl.ds(0, rows_per_w)])
        pltpu.sync_copy(
            q_hbm.at[pl.ds((b * 3 + 1) * NPOINT + s0, rows_per_w)],
            qy_v.at[pl.ds(0, rows_per_w)])
        pltpu.sync_copy(
            q_hbm.at[pl.ds((b * 3 + 2) * NPOINT + s0, rows_per_w)],
            qz_v.at[pl.ds(0, rows_per_w)])
        base = b * 8192
        wbase = wid * rows_per_w
        iota = lax.iota(jnp.int32, _L)
        big = jnp.int32(1 << 30)

        def scan_row(r, idx_v):
            qx = jnp.full((_L,), qx_v[pl.ds(r, _L)][0], jnp.float32)
            qy = jnp.full((_L,), qy_v[pl.ds(r, _L)][0], jnp.float32)
            qz = jnp.full((_L,), qz_v[pl.ds(r, _L)][0], jnp.float32)

            def cond(st):
                return jnp.logical_and(st[1] < NSAMPLE, st[0] < n_steps // 2)

            def wbody(st):
                step, cnt = st
                off = step * (2 * _L)
                cc = cnt
                for h in range(2):
                    o = off + h * _L
                    dx = xx_v[pl.ds(o, _L)] - qx
                    dy = xy_v[pl.ds(o, _L)] - qy
                    dz = xz_v[pl.ds(o, _L)] - qz
                    d2 = (dx * dx + dy * dy) + dz * dz
                    msk = d2 < r2
                    cols = iota + (o + base)
                    plsc.store_compressed(buf_v.at[pl.ds(cc, _L)], cols,
                                          mask=msk)
                    cc = cc + plsc.all_reduce_population_count(msk)[0]
                return (step + 1, cc)

            st = lax.while_loop(cond, wbody, (jnp.int32(0), jnp.int32(0)))
            cnt = st[1]
            first = buf_v[pl.ds(0, _L)][0]
            for j in range(NSAMPLE // _L):
                v = buf_v[pl.ds(_L * j, _L)]
                lane = iota + _L * j
                idx_v[pl.ds(_L * j, _L)] = jnp.where(lane < cnt, v, first)

        for kk in range(_KBUF):
            scan_row(jnp.int32(kk), idx_vs[kk])
            pltpu.async_copy(g_hbm.at[idx_vs[kk]], rows_vs[kk], gsems[kk])

        def outer_body(o, _):
            for kk in range(_KBUF):
                row = o * _KBUF + kk
                pltpu.make_async_copy(
                    g_hbm.at[idx_vs[kk]], rows_vs[kk], gsems[kk]).wait()
                wr = pltpu.async_copy(
                    rows_vs[kk], out_hbm.at[wbase + row - _KBUF], wsems[kk])
                scan_row(row, idx_vs[kk])
                wr.wait()
                pltpu.async_copy(g_hbm.at[idx_vs[kk]], rows_vs[kk], gsems[kk])
            return 0

        lax.fori_loop(1, rows_per_w // _KBUF, outer_body, 0)
        for kk in range(_KBUF):
            row = rows_per_w - _KBUF + kk
            pltpu.make_async_copy(
                g_hbm.at[idx_vs[kk]], rows_vs[kk], gsems[kk]).wait()
            pltpu.sync_copy(rows_vs[kk], out_hbm.at[wbase + row])

    return k(xyz_flat, new_xyz_flat, g_flat)


def _mlp_body(g_ref, q_ref, w1_ref, w2_ref, b2_ref, w3_ref, b3_ref, out_ref):
    s = jnp.sqrt(jnp.float32(1.0 + BN_EPS))
    g = g_ref[...]
    q = q_ref[...]
    w1x = w1_ref[...][:, 0:3]
    t = jnp.dot(q, w1x.T, preferred_element_type=jnp.float32)
    h1 = jax.nn.relu((g - t[:, None, :]) / s)
    h1 = h1.reshape(256 * NSAMPLE, 64)
    y2 = jnp.dot(h1, w2_ref[...].T, preferred_element_type=jnp.float32)
    h2 = jax.nn.relu((y2 + b2_ref[...]) / s)
    y3 = jnp.dot(h2, w3_ref[...].T, preferred_element_type=jnp.float32)
    h3 = jax.nn.relu((y3 + b3_ref[...]) / s)
    m = jnp.max(h3.reshape(256, NSAMPLE, 128), axis=1)
    out_ref[0] = m.T


def _mlp(gathered, new_xyz_flat, W1, W2, b2, W3, b3, B):
    return pl.pallas_call(
        _mlp_body,
        grid=(B, NPOINT // 256),
        in_specs=[
            pl.BlockSpec((256, NSAMPLE, 64),
                         lambda b, j: (b * (NPOINT // 256) + j, 0, 0)),
            pl.BlockSpec((256, 3), lambda b, j: (b * (NPOINT // 256) + j, 0)),
            pl.BlockSpec((64, 35), lambda b, j: (0, 0)),
            pl.BlockSpec((64, 64), lambda b, j: (0, 0)),
            pl.BlockSpec((1, 64), lambda b, j: (0, 0)),
            pl.BlockSpec((128, 64), lambda b, j: (0, 0)),
            pl.BlockSpec((1, 128), lambda b, j: (0, 0)),
        ],
        out_specs=pl.BlockSpec((1, 128, 256), lambda b, j: (b, 0, j)),
        out_shape=jax.ShapeDtypeStruct((B, 128, NPOINT), jnp.float32),
    )(gathered, new_xyz_flat, W1, W2, b2.reshape(1, 64), W3,
      b3.reshape(1, 128))


def kernel(xyz, features, W1, b1, W2, b2, W3, b3):
    B, N, _ = xyz.shape
    xyz_soa = jnp.transpose(xyz, (0, 2, 1))
    xyz_t3 = xyz_soa.reshape(B, 3, N // 128, 128)

    new_xyz = _fps(xyz_t3, xyz)
    G = _gtable(xyz, features, W1, b1)
    new_xyz_soa = jnp.transpose(new_xyz, (0, 2, 1))
    gathered = _ballgather(xyz_soa.reshape(B * 3 * N),
                           new_xyz_soa.reshape(B * 3 * NPOINT),
                           G.reshape(B * N, 64))
    new_features = _mlp(gathered, new_xyz.reshape(B * NPOINT, 3),
                        W1, W2, b2, W3, b3, B)
    return new_xyz, new_features


# speedup vs baseline: 1.1573x; 1.0120x over previous
import functools

import jax
import jax.numpy as jnp
from jax import lax
from jax.experimental import pallas as pl
from jax.experimental.pallas import tpu as pltpu
from jax.experimental.pallas import tpu_sc as plsc

NPOINT = 2048
NSAMPLE = 32
RADIUS = 0.2
BN_EPS = 1e-5

_NC = 2
_NS = 16
_L = 16
_KBUF = 4


def _fps_body(xyzt_ref, xyzn_ref, out_ref):
    X0, Y0, Z0 = xyzt_ref[0, 0], xyzt_ref[0, 1], xyzt_ref[0, 2]
    X1, Y1, Z1 = xyzt_ref[1, 0], xyzt_ref[1, 1], xyzt_ref[1, 2]
    iota2df = (lax.broadcasted_iota(jnp.int32, (64, 128), 0) * 128
               + lax.broadcasted_iota(jnp.int32, (64, 128), 1)
               ).astype(jnp.float32)
    out_ref[0, 0:1, :] = xyzn_ref[0, 0:1, :]
    out_ref[1, 0:1, :] = xyzn_ref[1, 0:1, :]
    bigf = jnp.float32(1e9)

    def body(i, carry):
        d0, d1, l0, l1 = carry
        c0 = xyzn_ref[0, pl.ds(l0, 1), :]
        c1 = xyzn_ref[1, pl.ds(l1, 1), :]
        dx0 = X0 - c0[0, 0]
        dy0 = Y0 - c0[0, 1]
        dz0 = Z0 - c0[0, 2]
        dx1 = X1 - c1[0, 0]
        dy1 = Y1 - c1[0, 1]
        dz1 = Z1 - c1[0, 2]
        d0 = jnp.minimum(d0, (dx0 * dx0 + dy0 * dy0) + dz0 * dz0)
        d1 = jnp.minimum(d1, (dx1 * dx1 + dy1 * dy1) + dz1 * dz1)
        m0 = jnp.max(d0, axis=(0, 1), keepdims=True)
        m1 = jnp.max(d1, axis=(0, 1), keepdims=True)
        n0 = jnp.min(jnp.where(d0 == m0, iota2df, bigf)).astype(jnp.int32)
        n1 = jnp.min(jnp.where(d1 == m1, iota2df, bigf)).astype(jnp.int32)
        out_ref[0, pl.ds(i, 1), :] = xyzn_ref[0, pl.ds(n0, 1), :]
        out_ref[1, pl.ds(i, 1), :] = xyzn_ref[1, pl.ds(n1, 1), :]
        return d0, d1, n0, n1

    dists0 = jnp.full((64, 128), 1e10, dtype=jnp.float32)
    lax.fori_loop(1, NPOINT, body,
                  (dists0, dists0, jnp.int32(0), jnp.int32(0)), unroll=2)


def _fps(xyz_t3, xyz):
    B = xyz.shape[0]
    return pl.pallas_call(
        _fps_body,
        out_shape=jax.ShapeDtypeStruct((B, NPOINT, 3), jnp.float32),
    )(xyz_t3, xyz)


def _gtable_body(xyz_ref, f_ref, w1_ref, b1_ref, out_ref):
    w1 = w1_ref[...]
    gx = jnp.dot(xyz_ref[0], w1[:, 0:3].T, preferred_element_type=jnp.float32)
    gf = lax.dot_general(f_ref[0], w1[:, 3:35],
                         (((0,), (1,)), ((), ())),
                         preferred_element_type=jnp.float32)
    out_ref[0] = (gx + gf) + b1_ref[...]


def _gtable(xyz, features, W1, b1):
    B, N, _ = xyz.shape
    return pl.pallas_call(
        _gtable_body,
        grid=(B, 8),
        in_specs=[
            pl.BlockSpec((1, 1024, 3), lambda b, j: (b, j, 0)),
            pl.BlockSpec((1, 32, 1024), lambda b, j: (b, 0, j)),
            pl.BlockSpec((64, 35), lambda b, j: (0, 0)),
            pl.BlockSpec((1, 64), lambda b, j: (0, 0)),
        ],
        out_specs=pl.BlockSpec((1, 1024, 64), lambda b, j: (b, j, 0)),
        out_shape=jax.ShapeDtypeStruct((B, N, 64), jnp.float32),
    )(xyz, features, W1, b1.reshape(1, 64))


def _ballgather(xyz_flat, new_xyz_flat, g_flat):
    B = xyz_flat.shape[0] // (3 * 8192)
    n_rows = B * NPOINT
    rows_per_w = n_rows // (_NC * _NS)
    w_per_b = _NS * _NC // B
    r2 = jnp.float32(RADIUS * RADIUS)
    n_steps = 8192 // _L
    mesh = plsc.VectorSubcoreMesh(core_axis_name="c", subcore_axis_name="s")

    @functools.partial(
        pl.kernel,
        out_type=jax.ShapeDtypeStruct((n_rows, NSAMPLE, 64), jnp.float32),
        mesh=mesh,
        compiler_params=pltpu.CompilerParams(needs_layout_passes=False,
                                             use_tc_tiling_on_sc=False),
        scratch_types=[
            pltpu.VMEM((8192,), jnp.float32),
            pltpu.VMEM((8192,), jnp.float32),
            pltpu.VMEM((8192,), jnp.float32),
            pltpu.VMEM((rows_per_w + _L,), jnp.float32),
            pltpu.VMEM((rows_per_w + _L,), jnp.float32),
            pltpu.VMEM((rows_per_w + _L,), jnp.float32),
            pltpu.VMEM((NSAMPLE + 2 * _L,), jnp.int32),
            [pltpu.VMEM((NSAMPLE,), jnp.int32) for _ in range(_KBUF)],
            [pltpu.VMEM((NSAMPLE, 64), jnp.float32) for _ in range(_KBUF)],
            [pltpu.SemaphoreType.DMA for _ in range(_KBUF)],
            [pltpu.SemaphoreType.DMA for _ in range(_KBUF)],
        ],
    )
    def k(xyz_hbm, q_hbm, g_hbm, out_hbm, xx_v, xy_v, xz_v, qx_v, qy_v, qz_v,
          buf_v, idx_vs, rows_vs, gsems, wsems):
        wid = lax.axis_index("s") * _NC + lax.axis_index("c")
        b = wid // w_per_b
        s0 = (wid % w_per_b) * rows_per_w
        pltpu.sync_copy(xyz_hbm.at[pl.ds((b * 3 + 0) * 8192, 8192)], xx_v)
        pltpu.sync_copy(xyz_hbm.at[pl.ds((b * 3 + 1) * 8192, 8192)], xy_v)
        pltpu.sync_copy(xyz_hbm.at[pl.ds((b * 3 + 2) * 8192, 8192)], xz_v)
        pltpu.sync_copy(
            q_hbm.at[pl.ds((b * 3 + 0) * NPOINT + s0, rows_per_w)],
            qx_v.at[pl.ds(0, rows_per_w)])
        pltpu.sync_copy(
            q_hbm.at[pl.ds((b * 3 + 1) * NPOINT + s0, rows_per_w)],
            qy_v.at[pl.ds(0, rows_per_w)])
        pltpu.sync_copy(
            q_hbm.at[pl.ds((b * 3 + 2) * NPOINT + s0, rows_per_w)],
            qz_v.at[pl.ds(0, rows_per_w)])
        base = b * 8192
        wbase = wid * rows_per_w
        iota = lax.iota(jnp.int32, _L)
        big = jnp.int32(1 << 30)

        def scan_row(r, idx_v):
            qx = jnp.full((_L,), qx_v[pl.ds(r, _L)][0], jnp.float32)
            qy = jnp.full((_L,), qy_v[pl.ds(r, _L)][0], jnp.float32)
            qz = jnp.full((_L,), qz_v[pl.ds(r, _L)][0], jnp.float32)

            def cond(st):
                return jnp.logical_and(st[1] < NSAMPLE, st[0] < n_steps // 2)

            def wbody(st):
                step, cnt = st
                off = step * (2 * _L)
                cc = cnt
                for h in range(2):
                    o = off + h * _L
                    dx = xx_v[pl.ds(o, _L)] - qx
                    dy = xy_v[pl.ds(o, _L)] - qy
                    dz = xz_v[pl.ds(o, _L)] - qz
                    d2 = (dx * dx + dy * dy) + dz * dz
                    msk = d2 < r2
                    cols = iota + (o + base)
                    plsc.store_compressed(buf_v.at[pl.ds(cc, _L)], cols,
                                          mask=msk)
                    cc = cc + plsc.all_reduce_population_count(msk)[0]
                return (step + 1, cc)

            st = lax.while_loop(cond, wbody, (jnp.int32(0), jnp.int32(0)))
            cnt = st[1]
            first = buf_v[pl.ds(0, _L)][0]
            for j in range(NSAMPLE // _L):
                v = buf_v[pl.ds(_L * j, _L)]
                lane = iota + _L * j
                idx_v[pl.ds(_L * j, _L)] = jnp.where(lane < cnt, v, first)

        for kk in range(_KBUF):
            scan_row(jnp.int32(kk), idx_vs[kk])
            pltpu.async_copy(g_hbm.at[idx_vs[kk]], rows_vs[kk], gsems[kk])

        def outer_body(o, _):
            for kk in range(_KBUF):
                row = o * _KBUF + kk
                pltpu.make_async_copy(
                    g_hbm.at[idx_vs[kk]], rows_vs[kk], gsems[kk]).wait()
                wr = pltpu.async_copy(
                    rows_vs[kk], out_hbm.at[wbase + row - _KBUF], wsems[kk])
                scan_row(row, idx_vs[kk])
                wr.wait()
                pltpu.async_copy(g_hbm.at[idx_vs[kk]], rows_vs[kk], gsems[kk])
            return 0

        lax.fori_loop(1, rows_per_w // _KBUF, outer_body, 0)
        for kk in range(_KBUF):
            row = rows_per_w - _KBUF + kk
            pltpu.make_async_copy(
                g_hbm.at[idx_vs[kk]], rows_vs[kk], gsems[kk]).wait()
            pltpu.sync_copy(rows_vs[kk], out_hbm.at[wbase + row])

    return k(xyz_flat, new_xyz_flat, g_flat)


def _mlp_body(g_ref, q_ref, w1_ref, w2_ref, b2_ref, w3_ref, b3_ref, out_ref):
    s = jnp.sqrt(jnp.float32(1.0 + BN_EPS))
    g = g_ref[...]
    q = q_ref[...]
    w1x = w1_ref[...][:, 0:3]
    t = jnp.dot(q, w1x.T, preferred_element_type=jnp.float32)
    h1 = jax.nn.relu((g - t[:, None, :]) / s)
    h1 = h1.reshape(256 * NSAMPLE, 64)
    y2 = jnp.dot(h1, w2_ref[...].T, preferred_element_type=jnp.float32)
    h2 = jax.nn.relu((y2 + b2_ref[...]) / s)
    y3 = jnp.dot(h2, w3_ref[...].T, preferred_element_type=jnp.float32)
    h3 = jax.nn.relu((y3 + b3_ref[...]) / s)
    m = jnp.max(h3.reshape(256, NSAMPLE, 128), axis=1)
    out_ref[0] = m.T


def _mlp(gathered, new_xyz_flat, W1, W2, b2, W3, b3, B):
    return pl.pallas_call(
        _mlp_body,
        grid=(B, NPOINT // 256),
        in_specs=[
            pl.BlockSpec((256, NSAMPLE, 64),
                         lambda b, j: (b * (NPOINT // 256) + j, 0, 0)),
            pl.BlockSpec((256, 3), lambda b, j: (b * (NPOINT // 256) + j, 0)),
            pl.BlockSpec((64, 35), lambda b, j: (0, 0)),
            pl.BlockSpec((64, 64), lambda b, j: (0, 0)),
            pl.BlockSpec((1, 64), lambda b, j: (0, 0)),
            pl.BlockSpec((128, 64), lambda b, j: (0, 0)),
            pl.BlockSpec((1, 128), lambda b, j: (0, 0)),
        ],
        out_specs=pl.BlockSpec((1, 128, 256), lambda b, j: (b, 0, j)),
        out_shape=jax.ShapeDtypeStruct((B, 128, NPOINT), jnp.float32),
    )(gathered, new_xyz_flat, W1, W2, b2.reshape(1, 64), W3,
      b3.reshape(1, 128))


def kernel(xyz, features, W1, b1, W2, b2, W3, b3):
    B, N, _ = xyz.shape
    xyz_soa = jnp.transpose(xyz, (0, 2, 1))
    xyz_t3 = xyz_soa.reshape(B, 3, N // 128, 128)

    new_xyz = _fps(xyz_t3, xyz)
    G = _gtable(xyz, features, W1, b1)
    new_xyz_soa = jnp.transpose(new_xyz, (0, 2, 1))
    gathered = _ballgather(xyz_soa.reshape(B * 3 * N),
                           new_xyz_soa.reshape(B * 3 * NPOINT),
                           G.reshape(B * N, 64))
    new_features = _mlp(gathered, new_xyz.reshape(B * NPOINT, 3),
                        W1, W2, b2, W3, b3, B)
    return new_xyz, new_features
